# baseline (speedup 1.0000x reference)
import functools

import jax
import jax.numpy as jnp
from jax import lax
from jax.experimental import pallas as pl
from jax.experimental.pallas import tpu as pltpu
from jax.experimental.pallas import tpu_sc as plsc

NUM_CLASSES = 100000
EMBED_DIM = 128
BATCH = 16384

NC = 2
NS = 16
NW = NC * NS
B_PER_W = BATCH // NW
CHUNKS = B_PER_W // 128


def _make_sc_gather():
    mesh = plsc.VectorSubcoreMesh(core_axis_name="c", subcore_axis_name="s")

    @functools.partial(
        pl.kernel,
        mesh=mesh,
        out_type=jax.ShapeDtypeStruct((BATCH, EMBED_DIM), jnp.float32),
        scratch_types=[
            pltpu.VMEM((CHUNKS, 128), jnp.int32),
            pltpu.VMEM((CHUNKS, 128, EMBED_DIM), jnp.float32),
            pltpu.SemaphoreType.DMA,
            pltpu.SemaphoreType.DMA,
        ],
    )
    def sc_gather(table_hbm, idx_hbm, out_hbm, idx_v, rows_v, gsem, wsem):
        wid = lax.axis_index("s") * NC + lax.axis_index("c")
        base = wid * B_PER_W
        pltpu.sync_copy(idx_hbm.at[wid], idx_v)
        gathers = [
            pltpu.async_copy(table_hbm.at[idx_v.at[j]], rows_v.at[j], gsem)
            for j in range(CHUNKS)
        ]
        writes = []
        for j in range(CHUNKS):
            gathers[j].wait()
            writes.append(
                pltpu.async_copy(
                    rows_v.at[j], out_hbm.at[pl.ds(base + j * 128, 128)], wsem
                )
            )
        for w in writes:
            w.wait()

    return sc_gather


_sc_gather = _make_sc_gather()


def _proj_body(x_ref, w_ref, b_ref, o_ref):
    x = x_ref[...]
    w = w_ref[...]
    acc = lax.dot_general(
        x, w,
        dimension_numbers=(((1,), (1,)), ((), ())),
        preferred_element_type=jnp.float32,
    )
    o_ref[...] = acc + b_ref[...]


def _project(x, W, b):
    BM = 1024
    grid = (BATCH // BM,)
    return pl.pallas_call(
        _proj_body,
        grid=grid,
        in_specs=[
            pl.BlockSpec((BM, EMBED_DIM), lambda i: (i, 0)),
            pl.BlockSpec((EMBED_DIM, EMBED_DIM), lambda i: (0, 0)),
            pl.BlockSpec((1, EMBED_DIM), lambda i: (0, 0)),
        ],
        out_specs=pl.BlockSpec((BM, EMBED_DIM), lambda i: (i, 0)),
        out_shape=jax.ShapeDtypeStruct((BATCH, EMBED_DIM), jnp.float32),
    )(x, W, b)


def kernel(class_labels, table, W, b):
    idx = class_labels.astype(jnp.int32).reshape(NW, CHUNKS, 128)
    x = _sc_gather(table, idx)
    out = _project(x, W, b.reshape(1, EMBED_DIM))
    return out[:, None, :]


# speedup vs baseline: 1.0100x; 1.0100x over previous
import functools

import jax
import jax.numpy as jnp
from jax import lax
from jax.experimental import pallas as pl
from jax.experimental.pallas import tpu as pltpu
from jax.experimental.pallas import tpu_sc as plsc

NUM_CLASSES = 100000
EMBED_DIM = 128
BATCH = 16384

NC = 2
NS = 16
NW = NC * NS

HALF = BATCH // 2
H_PER_W = HALF // NW
H_CHUNKS = H_PER_W // 128


def _make_sc_gather(rows):
    per_w = rows // NW
    chunks = per_w // 128
    mesh = plsc.VectorSubcoreMesh(core_axis_name="c", subcore_axis_name="s")

    @functools.partial(
        pl.kernel,
        mesh=mesh,
        out_type=jax.ShapeDtypeStruct((rows, EMBED_DIM), jnp.float32),
        scratch_types=[
            pltpu.VMEM((chunks, 128), jnp.int32),
            pltpu.VMEM((chunks, 128, EMBED_DIM), jnp.float32),
            pltpu.SemaphoreType.DMA,
            pltpu.SemaphoreType.DMA,
        ],
    )
    def sc_gather(table_hbm, idx_hbm, out_hbm, idx_v, rows_v, gsem, wsem):
        wid = lax.axis_index("s") * NC + lax.axis_index("c")
        base = wid * per_w
        pltpu.sync_copy(idx_hbm.at[wid], idx_v)
        gathers = [
            pltpu.async_copy(table_hbm.at[idx_v.at[j]], rows_v.at[j], gsem)
            for j in range(chunks)
        ]
        writes = []
        for j in range(chunks):
            gathers[j].wait()
            writes.append(
                pltpu.async_copy(
                    rows_v.at[j], out_hbm.at[pl.ds(base + j * 128, 128)], wsem
                )
            )
        for w in writes:
            w.wait()

    return sc_gather


_sc_gather_half = _make_sc_gather(HALF)

BM = 1024


def _proj_first_body(x_ref, w_ref, b_ref, o_ref):
    acc = lax.dot_general(
        x_ref[...], w_ref[...],
        dimension_numbers=(((1,), (1,)), ((), ())),
        preferred_element_type=jnp.float32,
    )
    o_ref[...] = acc + b_ref[...]


def _proj_first(x, W, b):
    return pl.pallas_call(
        _proj_first_body,
        grid=(HALF // BM,),
        in_specs=[
            pl.BlockSpec((BM, EMBED_DIM), lambda i: (i, 0)),
            pl.BlockSpec((EMBED_DIM, EMBED_DIM), lambda i: (0, 0)),
            pl.BlockSpec((1, EMBED_DIM), lambda i: (0, 0)),
        ],
        out_specs=pl.BlockSpec((BM, EMBED_DIM), lambda i: (i, 0)),
        out_shape=jax.ShapeDtypeStruct((BATCH, EMBED_DIM), jnp.float32),
    )(x, W, b)


def _proj_second_body(x_ref, w_ref, b_ref, prev_ref, o_ref):
    del prev_ref
    acc = lax.dot_general(
        x_ref[...], w_ref[...],
        dimension_numbers=(((1,), (1,)), ((), ())),
        preferred_element_type=jnp.float32,
    )
    o_ref[...] = acc + b_ref[...]


def _proj_second(x, W, b, prev):
    nblk = HALF // BM
    return pl.pallas_call(
        _proj_second_body,
        grid=(nblk,),
        in_specs=[
            pl.BlockSpec((BM, EMBED_DIM), lambda i: (i, 0)),
            pl.BlockSpec((EMBED_DIM, EMBED_DIM), lambda i: (0, 0)),
            pl.BlockSpec((1, EMBED_DIM), lambda i: (0, 0)),
            pl.BlockSpec((8, EMBED_DIM), lambda i: (0, 0)),
        ],
        out_specs=pl.BlockSpec((BM, EMBED_DIM), lambda i: (i + nblk, 0)),
        out_shape=jax.ShapeDtypeStruct((BATCH, EMBED_DIM), jnp.float32),
        input_output_aliases={3: 0},
    )(x, W, b, prev)


def kernel(class_labels, table, W, b):
    idx = class_labels.astype(jnp.int32).reshape(2, NW, H_CHUNKS, 128)
    x0 = _sc_gather_half(table, idx[0])
    x1 = _sc_gather_half(table, idx[1])
    b2 = b.reshape(1, EMBED_DIM)
    out = _proj_first(x0, W, b2)
    out = _proj_second(x1, W, b2, out)
    return out[:, None, :]


# speedup vs baseline: 1.0790x; 1.0683x over previous
import functools

import jax
import jax.numpy as jnp
from jax import lax
from jax.experimental import pallas as pl
from jax.experimental.pallas import tpu as pltpu
from jax.experimental.pallas import tpu_sc as plsc

NUM_CLASSES = 100000
EMBED_DIM = 128
BATCH = 16384

NC = 2
NS = 16
NW = NC * NS

HALF = BATCH // 2
H_PER_W = HALF // NW
H_CHUNKS = H_PER_W // 128


def _make_sc_gather(rows, offset):
    per_w = rows // NW
    chunks = per_w // 128
    mesh = plsc.VectorSubcoreMesh(core_axis_name="c", subcore_axis_name="s")

    @functools.partial(
        pl.kernel,
        mesh=mesh,
        out_type=jax.ShapeDtypeStruct((rows, EMBED_DIM), jnp.float32),
        scratch_types=[
            pltpu.VMEM((per_w,), jnp.int32),
            pltpu.VMEM((chunks, 128, EMBED_DIM), jnp.float32),
            pltpu.SemaphoreType.DMA,
            pltpu.SemaphoreType.DMA,
        ],
    )
    def sc_gather(table_hbm, idx_hbm, out_hbm, idx_v, rows_v, gsem, wsem):
        wid = lax.axis_index("s") * NC + lax.axis_index("c")
        base = wid * per_w
        pltpu.sync_copy(idx_hbm.at[pl.ds(offset + base, per_w)], idx_v)
        gathers = [
            pltpu.async_copy(
                table_hbm.at[idx_v.at[pl.ds(j * 128, 128)]], rows_v.at[j], gsem
            )
            for j in range(chunks)
        ]
        writes = []
        for j in range(chunks):
            gathers[j].wait()
            writes.append(
                pltpu.async_copy(
                    rows_v.at[j], out_hbm.at[pl.ds(base + j * 128, 128)], wsem
                )
            )
        for w in writes:
            w.wait()

    return sc_gather


_sc_gather_lo = _make_sc_gather(HALF, 0)
_sc_gather_hi = _make_sc_gather(HALF, HALF)

BM = 2048


def _proj_first_body(x_ref, w_ref, b_ref, o_ref):
    acc = lax.dot_general(
        x_ref[...], w_ref[...],
        dimension_numbers=(((1,), (1,)), ((), ())),
        preferred_element_type=jnp.float32,
    )
    o_ref[...] = acc + b_ref[...]


def _proj_first(x, W, b):
    return pl.pallas_call(
        _proj_first_body,
        grid=(HALF // BM,),
        in_specs=[
            pl.BlockSpec((BM, EMBED_DIM), lambda i: (i, 0)),
            pl.BlockSpec((EMBED_DIM, EMBED_DIM), lambda i: (0, 0)),
            pl.BlockSpec((1, EMBED_DIM), lambda i: (0, 0)),
        ],
        out_specs=pl.BlockSpec((BM, EMBED_DIM), lambda i: (i, 0)),
        out_shape=jax.ShapeDtypeStruct((BATCH, EMBED_DIM), jnp.float32),
    )(x, W, b)


def _proj_second_body(x_ref, w_ref, b_ref, prev_ref, o_ref):
    del prev_ref
    acc = lax.dot_general(
        x_ref[...], w_ref[...],
        dimension_numbers=(((1,), (1,)), ((), ())),
        preferred_element_type=jnp.float32,
    )
    o_ref[...] = acc + b_ref[...]


def _proj_second(x, W, b, prev):
    nblk = HALF // BM
    return pl.pallas_call(
        _proj_second_body,
        grid=(nblk,),
        in_specs=[
            pl.BlockSpec((BM, EMBED_DIM), lambda i: (i, 0)),
            pl.BlockSpec((EMBED_DIM, EMBED_DIM), lambda i: (0, 0)),
            pl.BlockSpec((1, EMBED_DIM), lambda i: (0, 0)),
            pl.BlockSpec((8, EMBED_DIM), lambda i: (0, 0)),
        ],
        out_specs=pl.BlockSpec((BM, EMBED_DIM), lambda i: (i + nblk, 0)),
        out_shape=jax.ShapeDtypeStruct((BATCH, EMBED_DIM), jnp.float32),
        input_output_aliases={3: 0},
    )(x, W, b, prev)


def kernel(class_labels, table, W, b):
    idx = class_labels.astype(jnp.int32)
    x0 = _sc_gather_lo(table, idx)
    x1 = _sc_gather_hi(table, idx)
    b2 = b.reshape(1, EMBED_DIM)
    out = _proj_first(x0, W, b2)
    out = _proj_second(x1, W, b2, out)
    return out[:, None, :]


# speedup vs baseline: 1.1007x; 1.0201x over previous
import functools

import jax
import jax.numpy as jnp
from jax import lax
from jax.experimental import pallas as pl
from jax.experimental.pallas import tpu as pltpu
from jax.experimental.pallas import tpu_sc as plsc

NUM_CLASSES = 100000
EMBED_DIM = 128
BATCH = 16384

NC = 2
NS = 16
NW = NC * NS

HALF = BATCH // 2
H_PER_W = HALF // NW
H_CHUNKS = H_PER_W // 128


def _make_sc_gather(rows, offset):
    per_w = rows // NW
    chunks = per_w // 128
    mesh = plsc.VectorSubcoreMesh(core_axis_name="c", subcore_axis_name="s")

    @functools.partial(
        pl.kernel,
        mesh=mesh,
        out_type=jax.ShapeDtypeStruct((rows, EMBED_DIM), jnp.float32),
        scratch_types=[
            pltpu.VMEM((per_w,), jnp.int32),
            pltpu.VMEM((chunks, 128, EMBED_DIM), jnp.float32),
            pltpu.SemaphoreType.DMA,
            pltpu.SemaphoreType.DMA,
        ],
    )
    def sc_gather(table_hbm, idx_hbm, out_hbm, idx_v, rows_v, gsem, wsem):
        wid = lax.axis_index("s") * NC + lax.axis_index("c")
        base = wid * per_w
        pltpu.sync_copy(idx_hbm.at[pl.ds(offset + base, per_w)], idx_v)
        gathers = [
            pltpu.async_copy(
                table_hbm.at[idx_v.at[pl.ds(j * 128, 128)]], rows_v.at[j], gsem
            )
            for j in range(chunks)
        ]
        writes = []
        for j in range(chunks):
            gathers[j].wait()
            writes.append(
                pltpu.async_copy(
                    rows_v.at[j], out_hbm.at[pl.ds(base + j * 128, 128)], wsem
                )
            )
        for w in writes:
            w.wait()

    return sc_gather


_sc_gather_lo = _make_sc_gather(HALF, 0)
_sc_gather_hi = _make_sc_gather(HALF, HALF)
_sc_gather_full = _make_sc_gather(BATCH, 0)

BM = 2048


def _proj_first_body(x_ref, w_ref, b_ref, o_ref):
    acc = lax.dot_general(
        x_ref[...], w_ref[...],
        dimension_numbers=(((1,), (1,)), ((), ())),
        preferred_element_type=jnp.float32,
    )
    o_ref[...] = acc + b_ref[...]


def _proj_first(x, W, b):
    return pl.pallas_call(
        _proj_first_body,
        grid=(HALF // BM,),
        in_specs=[
            pl.BlockSpec((BM, EMBED_DIM), lambda i: (i, 0)),
            pl.BlockSpec((EMBED_DIM, EMBED_DIM), lambda i: (0, 0)),
            pl.BlockSpec((1, EMBED_DIM), lambda i: (0, 0)),
        ],
        out_specs=pl.BlockSpec((BM, EMBED_DIM), lambda i: (i, 0)),
        out_shape=jax.ShapeDtypeStruct((BATCH, EMBED_DIM), jnp.float32),
    )(x, W, b)


def _proj_second_body(x_ref, w_ref, b_ref, prev_ref, o_ref):
    del prev_ref
    acc = lax.dot_general(
        x_ref[...], w_ref[...],
        dimension_numbers=(((1,), (1,)), ((), ())),
        preferred_element_type=jnp.float32,
    )
    o_ref[...] = acc + b_ref[...]


def _proj_second(x, W, b, prev):
    nblk = HALF // BM
    return pl.pallas_call(
        _proj_second_body,
        grid=(nblk,),
        in_specs=[
            pl.BlockSpec((BM, EMBED_DIM), lambda i: (i, 0)),
            pl.BlockSpec((EMBED_DIM, EMBED_DIM), lambda i: (0, 0)),
            pl.BlockSpec((1, EMBED_DIM), lambda i: (0, 0)),
            pl.BlockSpec((8, EMBED_DIM), lambda i: (0, 0)),
        ],
        out_specs=pl.BlockSpec((BM, EMBED_DIM), lambda i: (i + nblk, 0)),
        out_shape=jax.ShapeDtypeStruct((BATCH, EMBED_DIM), jnp.float32),
        input_output_aliases={3: 0},
    )(x, W, b, prev)


def _proj_full(x, W, b):
    return pl.pallas_call(
        _proj_first_body,
        grid=(BATCH // BM,),
        in_specs=[
            pl.BlockSpec((BM, EMBED_DIM), lambda i: (i, 0)),
            pl.BlockSpec((EMBED_DIM, EMBED_DIM), lambda i: (0, 0)),
            pl.BlockSpec((1, EMBED_DIM), lambda i: (0, 0)),
        ],
        out_specs=pl.BlockSpec((BM, EMBED_DIM), lambda i: (i, 0)),
        out_shape=jax.ShapeDtypeStruct((BATCH, EMBED_DIM), jnp.float32),
    )(x, W, b)


def kernel(class_labels, table, W, b):
    idx = class_labels.astype(jnp.int32)
    x = _sc_gather_full(table, idx)
    out = _proj_full(x, W, b.reshape(1, EMBED_DIM))
    return out[:, None, :]


# speedup vs baseline: 1.1844x; 1.0761x over previous
import functools

import jax
import jax.numpy as jnp
from jax import lax
from jax.experimental import pallas as pl
from jax.experimental.pallas import tpu as pltpu
from jax.experimental.pallas import tpu_sc as plsc

NUM_CLASSES = 100000
EMBED_DIM = 128
BATCH = 16384

NC = 2
NS = 16
NW = NC * NS

HALF = BATCH // 2
H_PER_W = HALF // NW
H_CHUNKS = H_PER_W // 128


def _make_sc_gather(rows, offset):
    per_w = rows // NW
    chunks = per_w // 128
    mesh = plsc.VectorSubcoreMesh(core_axis_name="c", subcore_axis_name="s")

    @functools.partial(
        pl.kernel,
        mesh=mesh,
        out_type=jax.ShapeDtypeStruct((rows, EMBED_DIM), jnp.float32),
        scratch_types=[
            pltpu.VMEM((per_w,), jnp.int32),
            pltpu.VMEM((chunks, 128, EMBED_DIM), jnp.float32),
            pltpu.SemaphoreType.DMA,
            pltpu.SemaphoreType.DMA,
        ],
    )
    def sc_gather(table_hbm, idx_hbm, out_hbm, idx_v, rows_v, gsem, wsem):
        wid = lax.axis_index("s") * NC + lax.axis_index("c")
        base = wid * per_w
        pltpu.sync_copy(idx_hbm.at[pl.ds(offset + base, per_w)], idx_v)
        gathers = [
            pltpu.async_copy(
                table_hbm.at[idx_v.at[pl.ds(j * 128, 128)]], rows_v.at[j], gsem
            )
            for j in range(chunks)
        ]
        writes = []
        for j in range(chunks):
            gathers[j].wait()
            writes.append(
                pltpu.async_copy(
                    rows_v.at[j], out_hbm.at[pl.ds(base + j * 128, 128)], wsem
                )
            )
        for w in writes:
            w.wait()

    return sc_gather


_sc_gather_lo = _make_sc_gather(HALF, 0)
_sc_gather_hi = _make_sc_gather(HALF, HALF)
_sc_gather_full = _make_sc_gather(BATCH, 0)

BM = 4096


def _proj_first_body(x_ref, w_ref, b_ref, o_ref):
    acc = lax.dot_general(
        x_ref[...], w_ref[...],
        dimension_numbers=(((1,), (1,)), ((), ())),
        preferred_element_type=jnp.float32,
    )
    o_ref[...] = acc + b_ref[...]


def _proj_first(x, W, b):
    return pl.pallas_call(
        _proj_first_body,
        grid=(HALF // BM,),
        in_specs=[
            pl.BlockSpec((BM, EMBED_DIM), lambda i: (i, 0)),
            pl.BlockSpec((EMBED_DIM, EMBED_DIM), lambda i: (0, 0)),
            pl.BlockSpec((1, EMBED_DIM), lambda i: (0, 0)),
        ],
        out_specs=pl.BlockSpec((BM, EMBED_DIM), lambda i: (i, 0)),
        out_shape=jax.ShapeDtypeStruct((BATCH, EMBED_DIM), jnp.float32),
    )(x, W, b)


def _proj_second_body(x_ref, w_ref, b_ref, prev_ref, o_ref):
    del prev_ref
    acc = lax.dot_general(
        x_ref[...], w_ref[...],
        dimension_numbers=(((1,), (1,)), ((), ())),
        preferred_element_type=jnp.float32,
    )
    o_ref[...] = acc + b_ref[...]


def _proj_second(x, W, b, prev):
    nblk = HALF // BM
    return pl.pallas_call(
        _proj_second_body,
        grid=(nblk,),
        in_specs=[
            pl.BlockSpec((BM, EMBED_DIM), lambda i: (i, 0)),
            pl.BlockSpec((EMBED_DIM, EMBED_DIM), lambda i: (0, 0)),
            pl.BlockSpec((1, EMBED_DIM), lambda i: (0, 0)),
            pl.BlockSpec((8, EMBED_DIM), lambda i: (0, 0)),
        ],
        out_specs=pl.BlockSpec((BM, EMBED_DIM), lambda i: (i + nblk, 0)),
        out_shape=jax.ShapeDtypeStruct((BATCH, EMBED_DIM), jnp.float32),
        input_output_aliases={3: 0},
    )(x, W, b, prev)


def _proj_full(x, W, b):
    return pl.pallas_call(
        _proj_first_body,
        grid=(BATCH // BM,),
        in_specs=[
            pl.BlockSpec((BM, EMBED_DIM), lambda i: (i, 0)),
            pl.BlockSpec((EMBED_DIM, EMBED_DIM), lambda i: (0, 0)),
            pl.BlockSpec((1, EMBED_DIM), lambda i: (0, 0)),
        ],
        out_specs=pl.BlockSpec((BM, EMBED_DIM), lambda i: (i, 0)),
        out_shape=jax.ShapeDtypeStruct((BATCH, EMBED_DIM), jnp.float32),
    )(x, W, b)


def kernel(class_labels, table, W, b):
    idx = class_labels.astype(jnp.int32)
    x = _sc_gather_full(table, idx)
    out = _proj_full(x, W, b.reshape(1, EMBED_DIM))
    return out[:, None, :]


# speedup vs baseline: 1.2347x; 1.0424x over previous
import functools

import jax
import jax.numpy as jnp
from jax import lax
from jax.experimental import pallas as pl
from jax.experimental.pallas import tpu as pltpu
from jax.experimental.pallas import tpu_sc as plsc

NUM_CLASSES = 100000
EMBED_DIM = 128
BATCH = 16384

NC = 2
NS = 16
NW = NC * NS

HALF = BATCH // 2
H_PER_W = HALF // NW
H_CHUNKS = H_PER_W // 128


def _make_sc_gather(rows, offset):
    per_w = rows // NW
    chunks = per_w // 128
    mesh = plsc.VectorSubcoreMesh(core_axis_name="c", subcore_axis_name="s")

    @functools.partial(
        pl.kernel,
        mesh=mesh,
        out_type=jax.ShapeDtypeStruct((rows, EMBED_DIM), jnp.float32),
        scratch_types=[
            pltpu.VMEM((per_w,), jnp.int32),
            pltpu.VMEM((chunks, 128, EMBED_DIM), jnp.float32),
            pltpu.SemaphoreType.DMA,
            pltpu.SemaphoreType.DMA,
        ],
    )
    def sc_gather(table_hbm, idx_hbm, out_hbm, idx_v, rows_v, gsem, wsem):
        wid = lax.axis_index("s") * NC + lax.axis_index("c")
        base = wid * per_w
        pltpu.sync_copy(idx_hbm.at[pl.ds(offset + base, per_w)], idx_v)
        gathers = [
            pltpu.async_copy(
                table_hbm.at[idx_v.at[pl.ds(j * 128, 128)]], rows_v.at[j], gsem
            )
            for j in range(chunks)
        ]
        writes = []
        for j in range(chunks):
            gathers[j].wait()
            writes.append(
                pltpu.async_copy(
                    rows_v.at[j], out_hbm.at[pl.ds(base + j * 128, 128)], wsem
                )
            )
        for w in writes:
            w.wait()

    return sc_gather


_sc_gather_lo = _make_sc_gather(HALF, 0)
_sc_gather_hi = _make_sc_gather(HALF, HALF)
_sc_gather_full = _make_sc_gather(BATCH, 0)

BM = 8192


def _proj_first_body(x_ref, w_ref, b_ref, o_ref):
    acc = lax.dot_general(
        x_ref[...], w_ref[...],
        dimension_numbers=(((1,), (1,)), ((), ())),
        preferred_element_type=jnp.float32,
    )
    o_ref[...] = acc + b_ref[...]


def _proj_first(x, W, b):
    return pl.pallas_call(
        _proj_first_body,
        grid=(HALF // BM,),
        in_specs=[
            pl.BlockSpec((BM, EMBED_DIM), lambda i: (i, 0)),
            pl.BlockSpec((EMBED_DIM, EMBED_DIM), lambda i: (0, 0)),
            pl.BlockSpec((1, EMBED_DIM), lambda i: (0, 0)),
        ],
        out_specs=pl.BlockSpec((BM, EMBED_DIM), lambda i: (i, 0)),
        out_shape=jax.ShapeDtypeStruct((BATCH, EMBED_DIM), jnp.float32),
    )(x, W, b)


def _proj_second_body(x_ref, w_ref, b_ref, prev_ref, o_ref):
    del prev_ref
    acc = lax.dot_general(
        x_ref[...], w_ref[...],
        dimension_numbers=(((1,), (1,)), ((), ())),
        preferred_element_type=jnp.float32,
    )
    o_ref[...] = acc + b_ref[...]


def _proj_second(x, W, b, prev):
    nblk = HALF // BM
    return pl.pallas_call(
        _proj_second_body,
        grid=(nblk,),
        in_specs=[
            pl.BlockSpec((BM, EMBED_DIM), lambda i: (i, 0)),
            pl.BlockSpec((EMBED_DIM, EMBED_DIM), lambda i: (0, 0)),
            pl.BlockSpec((1, EMBED_DIM), lambda i: (0, 0)),
            pl.BlockSpec((8, EMBED_DIM), lambda i: (0, 0)),
        ],
        out_specs=pl.BlockSpec((BM, EMBED_DIM), lambda i: (i + nblk, 0)),
        out_shape=jax.ShapeDtypeStruct((BATCH, EMBED_DIM), jnp.float32),
        input_output_aliases={3: 0},
    )(x, W, b, prev)


def _proj_full(x, W, b):
    return pl.pallas_call(
        _proj_first_body,
        grid=(BATCH // BM,),
        in_specs=[
            pl.BlockSpec((BM, EMBED_DIM), lambda i: (i, 0)),
            pl.BlockSpec((EMBED_DIM, EMBED_DIM), lambda i: (0, 0)),
            pl.BlockSpec((1, EMBED_DIM), lambda i: (0, 0)),
        ],
        out_specs=pl.BlockSpec((BM, EMBED_DIM), lambda i: (i, 0)),
        out_shape=jax.ShapeDtypeStruct((BATCH, EMBED_DIM), jnp.float32),
    )(x, W, b)


def kernel(class_labels, table, W, b):
    idx = class_labels.astype(jnp.int32)
    x = _sc_gather_full(table, idx)
    out = _proj_full(x, W, b.reshape(1, EMBED_DIM))
    return out[:, None, :]
